# Initial kernel scaffold; baseline (speedup 1.0000x reference)
import jax
import jax.numpy as jnp
from jax.experimental import pallas as pl


def kernel(x, emb, W1, b1, W2, b2):
    raise NotImplementedError("write your pallas kernel here")


# speedup vs baseline: 6.5522x; 6.5522x over previous
import jax
import jax.numpy as jnp
from jax import lax
from jax.experimental import pallas as pl
from jax.experimental.pallas import tpu as pltpu
from jax.experimental.pallas import tpu_sc as plsc

B = 4096
L = 200
EMB = 100
EMB_P = 104
HID = 64
OUT = 5

NC = 2
NS = 16
NW = NC * NS
ROWS_PER_W = B // NW
HALF = 100
CHUNKS_PER_W = 2 * ROWS_PER_W

OFFS = (0, 16, 32, 48, 64, 80, 84)


def _pool_body(x2_hbm, emb_hbm, out_hbm, idx_v, buf_v, acc_v, sem0, sem1):
    cid = lax.axis_index("c")
    sid = lax.axis_index("s")
    wid = sid * NC + cid
    base2 = wid * CHUNKS_PER_W
    baseb = wid * ROWS_PER_W

    pltpu.sync_copy(x2_hbm.at[pl.ds(base2, CHUNKS_PER_W)], idx_v)

    sems = (sem0, sem1)

    def issue(b, p):
        c = 2 * b
        pltpu.async_copy(emb_hbm.at[idx_v.at[c]], buf_v.at[2 * p], sems[p])
        pltpu.async_copy(emb_hbm.at[idx_v.at[c + 1]], buf_v.at[2 * p + 1], sems[p])

    def wait(p):
        pltpu.make_async_copy(emb_hbm.at[idx_v.at[0]], buf_v.at[2 * p], sems[p]).wait()
        pltpu.make_async_copy(emb_hbm.at[idx_v.at[0]], buf_v.at[2 * p + 1], sems[p]).wait()

    issue(0, 0)

    def outer(bb, carry):
        for p in (0, 1):
            b = 2 * bb + p

            @pl.when(b + 1 < ROWS_PER_W)
            def _():
                issue(b + 1, 1 - p)

            wait(p)

            b0 = buf_v.at[2 * p]
            b1 = buf_v.at[2 * p + 1]

            def rbody(r, accs):
                out = []
                for k, off in enumerate(OFFS):
                    v = b0[r, pl.ds(off, 16)] + b1[r, pl.ds(off, 16)]
                    out.append(accs[k] + v)
                return tuple(out)

            z = jnp.zeros((16,), jnp.float32)
            accs = lax.fori_loop(0, HALF, rbody, (z,) * len(OFFS))
            for k, off in enumerate(OFFS):
                acc_v[b, pl.ds(off, 16)] = accs[k]
        return carry

    lax.fori_loop(0, ROWS_PER_W // 2, outer, 0)

    pltpu.sync_copy(acc_v, out_hbm.at[pl.ds(baseb, ROWS_PER_W)])


def _pool(x2, emb):
    f = pl.kernel(
        _pool_body,
        out_type=jax.ShapeDtypeStruct((B, EMB), jnp.float32),
        mesh=plsc.VectorSubcoreMesh(core_axis_name="c", subcore_axis_name="s"),
        scratch_types=[
            pltpu.VMEM((CHUNKS_PER_W, HALF), jnp.int32),
            pltpu.VMEM((4, HALF, EMB_P), jnp.float32),
            pltpu.VMEM((ROWS_PER_W, EMB), jnp.float32),
            pltpu.SemaphoreType.DMA,
            pltpu.SemaphoreType.DMA,
        ],
        compiler_params=pltpu.CompilerParams(use_tc_tiling_on_sc=False),
    )
    return f(x2, emb)


def _mlp_body(s_ref, w1_ref, b1_ref, w2_ref, b2_ref, o_ref):
    h = s_ref[...] * (1.0 / L)
    h = jnp.dot(h, w1_ref[...], preferred_element_type=jnp.float32) + b1_ref[...]
    h = jnp.maximum(h, 0.0)
    logits = jnp.dot(h, w2_ref[...], preferred_element_type=jnp.float32) + b2_ref[...]
    m = jnp.max(logits, axis=1, keepdims=True)
    e = jnp.exp(logits - m)
    o_ref[...] = e / jnp.sum(e, axis=1, keepdims=True)


def _mlp(sums, W1, b1, W2, b2):
    return pl.pallas_call(
        _mlp_body,
        out_shape=jax.ShapeDtypeStruct((B, OUT), jnp.float32),
    )(sums, W1, b1.reshape(1, HID), W2, b2.reshape(1, OUT))


def kernel(x, emb, W1, b1, W2, b2):
    x2 = x.reshape(B * 2, HALF)
    emb_p = jnp.pad(emb, ((0, 0), (0, EMB_P - EMB)))
    sums = _pool(x2, emb_p)
    return _mlp(sums, W1, b1, W2, b2)


# speedup vs baseline: 7.3979x; 1.1291x over previous
import jax
import jax.numpy as jnp
from jax import lax
from jax.experimental import pallas as pl
from jax.experimental.pallas import tpu as pltpu
from jax.experimental.pallas import tpu_sc as plsc

B = 4096
L = 200
EMB = 100
EMB_P = 128
HID = 64
OUT = 5

NC = 2
NS = 16
NW = NC * NS
ROWS_PER_W = B // NW
HALF = 100
CHUNKS_PER_W = 2 * ROWS_PER_W

OFFS = (0, 16, 32, 48, 64, 80, 84)


def _pool_body(x2_hbm, emb_hbm, out_hbm, idx_v, buf_v, acc_v, sem0, sem1):
    cid = lax.axis_index("c")
    sid = lax.axis_index("s")
    wid = sid * NC + cid
    base2 = wid * CHUNKS_PER_W
    baseb = wid * ROWS_PER_W

    pltpu.sync_copy(x2_hbm.at[pl.ds(base2, CHUNKS_PER_W)], idx_v)

    sems = (sem0, sem1)

    def issue(b, p):
        c = 2 * b
        pltpu.async_copy(emb_hbm.at[idx_v.at[c]], buf_v.at[2 * p], sems[p])
        pltpu.async_copy(emb_hbm.at[idx_v.at[c + 1]], buf_v.at[2 * p + 1], sems[p])

    def wait(p):
        pltpu.make_async_copy(emb_hbm.at[idx_v.at[0]], buf_v.at[2 * p], sems[p]).wait()
        pltpu.make_async_copy(emb_hbm.at[idx_v.at[0]], buf_v.at[2 * p + 1], sems[p]).wait()

    issue(0, 0)

    def outer(bb, carry):
        for p in (0, 1):
            b = 2 * bb + p

            @pl.when(b + 1 < ROWS_PER_W)
            def _():
                issue(b + 1, 1 - p)

            wait(p)

            b0 = buf_v.at[2 * p]
            b1 = buf_v.at[2 * p + 1]

            def rbody(r, accs):
                out = []
                for k, off in enumerate(OFFS):
                    v = b0[r, pl.ds(off, 16)] + b1[r, pl.ds(off, 16)]
                    out.append(accs[k] + v)
                return tuple(out)

            z = jnp.zeros((16,), jnp.float32)
            accs = lax.fori_loop(0, HALF, rbody, (z,) * len(OFFS))
            for k, off in enumerate(OFFS):
                acc_v[b, pl.ds(off, 16)] = accs[k]
        return carry

    lax.fori_loop(0, ROWS_PER_W // 2, outer, 0)

    pltpu.sync_copy(acc_v, out_hbm.at[pl.ds(baseb, ROWS_PER_W)])


def _pool(x2, emb):
    f = pl.kernel(
        _pool_body,
        out_type=jax.ShapeDtypeStruct((B, EMB), jnp.float32),
        mesh=plsc.VectorSubcoreMesh(core_axis_name="c", subcore_axis_name="s"),
        scratch_types=[
            pltpu.VMEM((CHUNKS_PER_W, HALF), jnp.int32),
            pltpu.VMEM((4, HALF, EMB_P), jnp.float32),
            pltpu.VMEM((ROWS_PER_W, EMB), jnp.float32),
            pltpu.SemaphoreType.DMA,
            pltpu.SemaphoreType.DMA,
        ],
        compiler_params=pltpu.CompilerParams(use_tc_tiling_on_sc=True),
    )
    return f(x2, emb)


def _mlp_body(s_ref, w1_ref, b1_ref, w2_ref, b2_ref, o_ref):
    h = s_ref[...] * (1.0 / L)
    h = jnp.dot(h, w1_ref[...], preferred_element_type=jnp.float32) + b1_ref[...]
    h = jnp.maximum(h, 0.0)
    logits = jnp.dot(h, w2_ref[...], preferred_element_type=jnp.float32) + b2_ref[...]
    m = jnp.max(logits, axis=1, keepdims=True)
    e = jnp.exp(logits - m)
    o_ref[...] = e / jnp.sum(e, axis=1, keepdims=True)


def _mlp(sums, W1, b1, W2, b2):
    return pl.pallas_call(
        _mlp_body,
        out_shape=jax.ShapeDtypeStruct((B, OUT), jnp.float32),
    )(sums, W1, b1.reshape(1, HID), W2, b2.reshape(1, OUT))


def kernel(x, emb, W1, b1, W2, b2):
    x2 = x.reshape(B * 2, HALF)
    emb_p = jnp.pad(emb, ((0, 0), (0, EMB_P - EMB)))
    sums = _pool(x2, emb_p)
    return _mlp(sums, W1, b1, W2, b2)


# speedup vs baseline: 10.3793x; 1.4030x over previous
import jax
import jax.numpy as jnp
from jax import lax
from jax.experimental import pallas as pl
from jax.experimental.pallas import tpu as pltpu
from jax.experimental.pallas import tpu_sc as plsc

B = 4096
L = 200
EMB = 100
EMB_P = 128
HID = 64
OUT = 5

NC = 2
NS = 16
NW = NC * NS
ROWS_PER_W = B // NW
HALF = 100
CHUNKS_PER_W = 2 * ROWS_PER_W

OFFS = (0, 16, 32, 48, 64, 80, 84)


def _pool_body(x2_hbm, emb_hbm, out_hbm, idx_v, buf_v, acc_v, sem0, sem1):
    cid = lax.axis_index("c")
    sid = lax.axis_index("s")
    wid = sid * NC + cid
    base2 = wid * CHUNKS_PER_W
    baseb = wid * ROWS_PER_W

    pltpu.sync_copy(x2_hbm.at[pl.ds(base2, CHUNKS_PER_W)], idx_v)

    sems = (sem0, sem1)

    def issue(b, p):
        c = 2 * b
        pltpu.async_copy(emb_hbm.at[idx_v.at[c]], buf_v.at[2 * p], sems[p])
        pltpu.async_copy(emb_hbm.at[idx_v.at[c + 1]], buf_v.at[2 * p + 1], sems[p])

    def wait(p):
        pltpu.make_async_copy(emb_hbm.at[idx_v.at[0]], buf_v.at[2 * p], sems[p]).wait()
        pltpu.make_async_copy(emb_hbm.at[idx_v.at[0]], buf_v.at[2 * p + 1], sems[p]).wait()

    issue(0, 0)

    def outer(bb, carry):
        for p in (0, 1):
            b = 2 * bb + p

            @pl.when(b + 1 < ROWS_PER_W)
            def _():
                issue(b + 1, 1 - p)

            wait(p)

            b0 = buf_v.at[2 * p]
            b1 = buf_v.at[2 * p + 1]

            def rbody(r, accs):
                out = []
                for k, off in enumerate(OFFS):
                    v = b0[r, pl.ds(off, 16)] + b1[r, pl.ds(off, 16)]
                    out.append(accs[k] + v)
                return tuple(out)

            z = jnp.zeros((16,), jnp.float32)
            accs = lax.fori_loop(0, HALF, rbody, (z,) * len(OFFS))
            for k, off in enumerate(OFFS):
                acc_v[b, pl.ds(off, 16)] = accs[k]
        return carry

    lax.fori_loop(0, ROWS_PER_W // 2, outer, 0)

    pltpu.sync_copy(acc_v, out_hbm.at[pl.ds(baseb, ROWS_PER_W)])


def _pool(x2, emb):
    f = pl.kernel(
        _pool_body,
        out_type=jax.ShapeDtypeStruct((B, EMB), jnp.float32),
        mesh=plsc.VectorSubcoreMesh(core_axis_name="c", subcore_axis_name="s"),
        scratch_types=[
            pltpu.VMEM((CHUNKS_PER_W, HALF), jnp.int32),
            pltpu.VMEM((4, HALF, EMB_P), jnp.float32),
            pltpu.VMEM((ROWS_PER_W, EMB), jnp.float32),
            pltpu.SemaphoreType.DMA,
            pltpu.SemaphoreType.DMA,
        ],
        compiler_params=pltpu.CompilerParams(use_tc_tiling_on_sc=True),
    )
    return f(x2, emb)


PAD_BLK = 4000


def _pad_body(e_ref, o_ref):
    o_ref[:, 0:EMB] = e_ref[...]


def _pad_table(emb):
    return pl.pallas_call(
        _pad_body,
        grid=(emb.shape[0] // PAD_BLK,),
        in_specs=[pl.BlockSpec((PAD_BLK, EMB), lambda i: (i, 0))],
        out_specs=pl.BlockSpec((PAD_BLK, EMB_P), lambda i: (i, 0)),
        out_shape=jax.ShapeDtypeStruct((emb.shape[0], EMB_P), jnp.float32),
    )(emb)


def _mlp_body(s_ref, w1_ref, b1_ref, w2_ref, b2_ref, o_ref):
    h = s_ref[...] * (1.0 / L)
    h = jnp.dot(h, w1_ref[...], preferred_element_type=jnp.float32) + b1_ref[...]
    h = jnp.maximum(h, 0.0)
    logits = jnp.dot(h, w2_ref[...], preferred_element_type=jnp.float32) + b2_ref[...]
    m = jnp.max(logits, axis=1, keepdims=True)
    e = jnp.exp(logits - m)
    o_ref[...] = e / jnp.sum(e, axis=1, keepdims=True)


def _mlp(sums, W1, b1, W2, b2):
    return pl.pallas_call(
        _mlp_body,
        out_shape=jax.ShapeDtypeStruct((B, OUT), jnp.float32),
    )(sums, W1, b1.reshape(1, HID), W2, b2.reshape(1, OUT))


def kernel(x, emb, W1, b1, W2, b2):
    x2 = x.reshape(B * 2, HALF)
    emb_p = _pad_table(emb)
    sums = _pool(x2, emb_p)
    return _mlp(sums, W1, b1, W2, b2)


# speedup vs baseline: 10.3818x; 1.0002x over previous
import jax
import jax.numpy as jnp
from jax import lax
from jax.experimental import pallas as pl
from jax.experimental.pallas import tpu as pltpu
from jax.experimental.pallas import tpu_sc as plsc

B = 4096
L = 200
EMB = 100
EMB_P = 128
HID = 64
OUT = 5

NC = 2
NS = 16
NW = NC * NS
ROWS_PER_W = B // NW
HALF = 100
CHUNKS_PER_W = 2 * ROWS_PER_W

OFFS = (0, 16, 32, 48, 64, 80, 84)


def _pool_body(x2_hbm, emb_hbm, out_hbm, idx_v, buf_v, acc_v, sem0, sem1):
    cid = lax.axis_index("c")
    sid = lax.axis_index("s")
    wid = sid * NC + cid
    base2 = wid * CHUNKS_PER_W
    baseb = wid * ROWS_PER_W

    pltpu.sync_copy(x2_hbm.at[pl.ds(base2, CHUNKS_PER_W)], idx_v)

    sems = (sem0, sem1)

    def issue(b, p):
        c = 2 * b
        pltpu.async_copy(emb_hbm.at[idx_v.at[c]], buf_v.at[2 * p], sems[p])
        pltpu.async_copy(emb_hbm.at[idx_v.at[c + 1]], buf_v.at[2 * p + 1], sems[p])

    def wait(p):
        pltpu.make_async_copy(emb_hbm.at[idx_v.at[0]], buf_v.at[2 * p], sems[p]).wait()
        pltpu.make_async_copy(emb_hbm.at[idx_v.at[0]], buf_v.at[2 * p + 1], sems[p]).wait()

    issue(0, 0)

    def outer(bb, carry):
        for p in (0, 1):
            b = 2 * bb + p

            @pl.when(b + 1 < ROWS_PER_W)
            def _():
                issue(b + 1, 1 - p)

            wait(p)

            b0 = buf_v.at[2 * p]
            b1 = buf_v.at[2 * p + 1]

            def rbody(i, accs):
                r = 4 * i
                out = list(accs)
                for u in range(4):
                    for k, off in enumerate(OFFS):
                        v = b0[r + u, pl.ds(off, 16)] + b1[r + u, pl.ds(off, 16)]
                        out[k] = out[k] + v
                return tuple(out)

            z = jnp.zeros((16,), jnp.float32)
            accs = lax.fori_loop(0, HALF // 4, rbody, (z,) * len(OFFS))
            for k, off in enumerate(OFFS):
                acc_v[b, pl.ds(off, 16)] = accs[k]
        return carry

    lax.fori_loop(0, ROWS_PER_W // 2, outer, 0)

    pltpu.sync_copy(acc_v, out_hbm.at[pl.ds(baseb, ROWS_PER_W)])


def _pool(x2, emb):
    f = pl.kernel(
        _pool_body,
        out_type=jax.ShapeDtypeStruct((B, EMB), jnp.float32),
        mesh=plsc.VectorSubcoreMesh(core_axis_name="c", subcore_axis_name="s"),
        scratch_types=[
            pltpu.VMEM((CHUNKS_PER_W, HALF), jnp.int32),
            pltpu.VMEM((4, HALF, EMB_P), jnp.float32),
            pltpu.VMEM((ROWS_PER_W, EMB), jnp.float32),
            pltpu.SemaphoreType.DMA,
            pltpu.SemaphoreType.DMA,
        ],
        compiler_params=pltpu.CompilerParams(use_tc_tiling_on_sc=True),
    )
    return f(x2, emb)


PAD_BLK = 4000


def _pad_body(e_ref, o_ref):
    o_ref[:, 0:EMB] = e_ref[...]


def _pad_table(emb):
    return pl.pallas_call(
        _pad_body,
        grid=(emb.shape[0] // PAD_BLK,),
        in_specs=[pl.BlockSpec((PAD_BLK, EMB), lambda i: (i, 0))],
        out_specs=pl.BlockSpec((PAD_BLK, EMB_P), lambda i: (i, 0)),
        out_shape=jax.ShapeDtypeStruct((emb.shape[0], EMB_P), jnp.float32),
    )(emb)


def _mlp_body(s_ref, w1_ref, b1_ref, w2_ref, b2_ref, o_ref):
    h = s_ref[...] * (1.0 / L)
    h = jnp.dot(h, w1_ref[...], preferred_element_type=jnp.float32) + b1_ref[...]
    h = jnp.maximum(h, 0.0)
    logits = jnp.dot(h, w2_ref[...], preferred_element_type=jnp.float32) + b2_ref[...]
    m = jnp.max(logits, axis=1, keepdims=True)
    e = jnp.exp(logits - m)
    o_ref[...] = e / jnp.sum(e, axis=1, keepdims=True)


def _mlp(sums, W1, b1, W2, b2):
    return pl.pallas_call(
        _mlp_body,
        out_shape=jax.ShapeDtypeStruct((B, OUT), jnp.float32),
    )(sums, W1, b1.reshape(1, HID), W2, b2.reshape(1, OUT))


def kernel(x, emb, W1, b1, W2, b2):
    x2 = x.reshape(B * 2, HALF)
    emb_p = _pad_table(emb)
    sums = _pool(x2, emb_p)
    return _mlp(sums, W1, b1, W2, b2)
